# baseline (speedup 1.0000x reference)
import functools

import jax
import jax.numpy as jnp
from jax import lax
from jax.experimental import pallas as pl
from jax.experimental.pallas import tpu as pltpu
from jax.experimental.pallas import tpu_sc as plsc

NUM_WORKERS = 32
NBUF = 4
S2PAD = 104
SEGS = ((0, 64), (64, 36))


def _make_gather(n_rows, s2pad, d):
    rows_per_w = n_rows // NUM_WORKERS
    mesh = plsc.VectorSubcoreMesh(core_axis_name="c", subcore_axis_name="s")

    @functools.partial(
        pl.kernel,
        out_type=jax.ShapeDtypeStruct((n_rows * s2pad, 2 * d), jnp.float32),
        mesh=mesh,
        scratch_types=(
            [pltpu.VMEM((rows_per_w, 128), jnp.int32),
             pltpu.VMEM((rows_per_w, 128), jnp.int32),
             pltpu.VMEM((NBUF, 112, d), jnp.float32)]
            + [pltpu.SemaphoreType.DMA] * (2 * NBUF)
        ),
        compiler_params=pltpu.CompilerParams(use_tc_tiling_on_sc=False),
    )
    def gather_kernel(idxe_hbm, idxo_hbm, table_hbm, out_hbm,
                      idxe_v, idxo_v, rows_v, *sems):
        gsem = sems[:NBUF]
        wsem = sems[NBUF:]
        wid = lax.axis_index("s") * 2 + lax.axis_index("c")
        row0 = wid * rows_per_w
        pltpu.sync_copy(idxe_hbm.at[pl.ds(row0, rows_per_w)], idxe_v)
        pltpu.sync_copy(idxo_hbm.at[pl.ds(row0, rows_per_w)], idxo_v)

        gdummy = table_hbm.at[pl.ds(0, 112)]
        wdummy = table_hbm.at[pl.ds(0, s2pad)]

        def fire(r, par, b):
            idx = idxe_v if par == 0 else idxo_v
            pltpu.async_copy(table_hbm.at[idx.at[r, pl.ds(0, 112)]],
                             rows_v.at[b], gsem[b])

        def put(r, par, b):
            pltpu.async_copy(rows_v.at[b, pl.ds(0, s2pad)],
                             out_hbm.at[pl.ds((row0 + r) * s2pad, s2pad),
                                        pl.ds(par * d, d)],
                             wsem[b])

        def drain(sem, b):
            dummy = gdummy if sem is gsem else wdummy
            pltpu.make_async_copy(dummy, rows_v.at[b, pl.ds(0, dummy.shape[0])],
                                  sem[b]).wait()

        for b in range(NBUF - 1):
            fire(b // 2, b % 2, b)

        n_chunks = 2 * rows_per_w

        def outer(p, carry):
            for k in range(NBUF):
                c = NBUF * p + k
                fb = (k + NBUF - 1) % NBUF

                @pl.when(c >= 1)
                def _():
                    drain(wsem, fb)

                @pl.when(c + NBUF - 1 < n_chunks)
                def _():
                    fire(2 * p + (k + NBUF - 1) // 2, (k + NBUF - 1) % 2, fb)

                drain(gsem, k)
                put(2 * p + k // 2, k % 2, k)
            return carry

        lax.fori_loop(0, n_chunks // NBUF, outer, 0, unroll=False)
        drain(wsem, (n_chunks - 1) % NBUF)

    return gather_kernel


def _transpose_block(x_ref, o_ref):
    nb, s2pad, nl = x_ref.shape
    x = x_ref[...]
    eye = (lax.broadcasted_iota(jnp.int32, (nb, nb), 0)
           == lax.broadcasted_iota(jnp.int32, (nb, nb), 1)
           ).astype(jnp.float32)
    xt = lax.dot_general(x, eye, (((0,), (0,)), ((), ())),
                         preferred_element_type=jnp.float32)
    o_ref[...] = xt


def _make_transpose(n_rows, s2pad, seq, d):
    bt = 2 * d
    return pl.pallas_call(
        _transpose_block,
        grid=(n_rows // bt,),
        in_specs=[pl.BlockSpec((bt, s2pad, 2 * d), lambda i: (i, 0, 0))],
        out_specs=pl.BlockSpec((s2pad, 2 * d, bt), lambda i: (0, 0, i)),
        out_shape=jax.ShapeDtypeStruct((s2pad, 2 * d, n_rows), jnp.float32),
    )


def kernel(code, embedding_table):
    b, s = code.shape
    v, d = embedding_table.shape
    assert b % NUM_WORKERS == 0 and s == 200 and d == 64
    idx = code.astype(jnp.int32)
    idxe = jnp.pad(idx[:, 0::2], ((0, 0), (0, 128 - s // 2)))
    idxo = jnp.pad(idx[:, 1::2], ((0, 0), (0, 128 - s // 2)))
    g = _make_gather(b, S2PAD, d)(idxe, idxo, embedding_table)
    g3 = g.reshape(b, S2PAD, 2 * d)
    t2 = _make_transpose(b, S2PAD, s, d)(g3)
    t3 = t2.reshape(2 * S2PAD, d, b)
    return jnp.transpose(t3, (2, 0, 1))[:, :s, :]


# speedup vs baseline: 3.0229x; 3.0229x over previous
import functools

import jax
import jax.numpy as jnp
from jax import lax
from jax.experimental import pallas as pl
from jax.experimental.pallas import tpu as pltpu
from jax.experimental.pallas import tpu_sc as plsc

NUM_WORKERS = 32
CHUNK = 128
NBUF = 5


def _make_gather(n_chunks, d):
    mesh = plsc.VectorSubcoreMesh(core_axis_name="c", subcore_axis_name="s")

    @functools.partial(
        pl.kernel,
        out_type=jax.ShapeDtypeStruct((NUM_WORKERS, n_chunks, CHUNK, d),
                                      jnp.float32),
        mesh=mesh,
        scratch_types=(
            [pltpu.VMEM((n_chunks, CHUNK), jnp.int32),
             pltpu.VMEM((NBUF, CHUNK, d), jnp.float32)]
            + [pltpu.SemaphoreType.DMA] * (2 * NBUF)
        ),
        compiler_params=pltpu.CompilerParams(use_tc_tiling_on_sc=False),
    )
    def gather_kernel(idx_hbm, table_hbm, out_hbm, idx_v, rows_v, *sems):
        gsem = sems[:NBUF]
        wsem = sems[NBUF:]
        wid = lax.axis_index("s") * 2 + lax.axis_index("c")
        pltpu.sync_copy(idx_hbm.at[wid], idx_v)

        dummy_src = table_hbm.at[pl.ds(0, CHUNK)]

        for b in range(NBUF - 1):
            pltpu.async_copy(table_hbm.at[idx_v.at[b]], rows_v.at[b], gsem[b])

        assert n_chunks % NBUF == 0
        n_outer = n_chunks // NBUF

        def outer(g0, carry):
            for i in range(NBUF):
                j = g0 * NBUF + i
                fb = (i + NBUF - 1) % NBUF

                @pl.when(j >= 1)
                def _():
                    pltpu.make_async_copy(dummy_src, rows_v.at[fb],
                                          wsem[fb]).wait()

                @pl.when(j + NBUF - 1 < n_chunks)
                def _():
                    pltpu.async_copy(
                        table_hbm.at[idx_v.at[j + NBUF - 1]],
                        rows_v.at[fb], gsem[fb])

                pltpu.make_async_copy(dummy_src, rows_v.at[i], gsem[i]).wait()
                pltpu.async_copy(rows_v.at[i], out_hbm.at[wid, j], wsem[i])
            return carry

        lax.fori_loop(0, n_outer, outer, 0, unroll=False)
        pltpu.make_async_copy(dummy_src, rows_v.at[(n_chunks - 1) % NBUF],
                              wsem[(n_chunks - 1) % NBUF]).wait()

    return gather_kernel


def kernel(code, embedding_table):
    b, s = code.shape
    v, d = embedding_table.shape
    total = b * s
    assert total % (NUM_WORKERS * CHUNK) == 0
    n_chunks = total // (NUM_WORKERS * CHUNK)
    idx = code.reshape(NUM_WORKERS, n_chunks, CHUNK).astype(jnp.int32)
    out = _make_gather(n_chunks, d)(idx, embedding_table)
    return out.reshape(b, s, d)
